# baseline (speedup 1.0000x reference)
import functools

import jax
import jax.numpy as jnp
from jax import lax
from jax.experimental import pallas as pl
from jax.experimental.pallas import tpu as pltpu
from jax.experimental.pallas import tpu_sc as plsc

N = 10000
E = 320000
D = 128
NC = 2
NS = 16
NW = NC * NS
K = 80
NB = E // (NW * K)
CB = 5
CHUNK = CB * K
NCH = NB // CB
ZR = 624


def _sc_body(x_hbm, src_hbm, dst_hbm, w_hbm, out_hbm,
             srcs, dsts, ws, rows, acc, sg, ss, si):
    cid = lax.axis_index("c")
    sid = lax.axis_index("s")
    wid = cid * NS + sid
    ebase = wid * (NB * K)

    zv = jnp.zeros((16,), jnp.float32)

    @plsc.parallel_loop(0, K)
    def _(i):
        for jj in range(D // 16):
            rows[0, i, pl.ds(jj * 16, 16)] = zv

    zb = sid * 625
    for q in range(625 // K):
        pltpu.sync_copy(rows.at[0], acc.at[pl.ds(zb + q * K, K)])
    if 625 % K:
        pltpu.sync_copy(rows.at[0, pl.ds(0, 625 % K)],
                        acc.at[pl.ds(zb + (625 // K) * K, 625 % K)])

    def start_chunk(c):
        s = lax.rem(c, 2)
        eb = ebase + c * CHUNK
        pltpu.async_copy(src_hbm.at[pl.ds(eb, CHUNK)],
                         srcs.at[pl.ds(s * CHUNK, CHUNK)], si)
        pltpu.async_copy(dst_hbm.at[pl.ds(eb, CHUNK)],
                         dsts.at[pl.ds(s * CHUNK, CHUNK)], si)
        pltpu.async_copy(w_hbm.at[pl.ds(eb, CHUNK)],
                         ws.at[pl.ds(s * CHUNK, CHUNK)], si)

    def wait_chunk(s):
        pltpu.make_async_copy(src_hbm.at[pl.ds(0, CHUNK)],
                              srcs.at[pl.ds(s * CHUNK, CHUNK)], si).wait()
        pltpu.make_async_copy(dst_hbm.at[pl.ds(0, CHUNK)],
                              dsts.at[pl.ds(s * CHUNK, CHUNK)], si).wait()
        pltpu.make_async_copy(w_hbm.at[pl.ds(0, CHUNK)],
                              ws.at[pl.ds(s * CHUNK, CHUNK)], si).wait()

    def start_gather(s, jc, b):
        pltpu.async_copy(x_hbm.at[srcs.at[pl.ds(s * CHUNK + jc * K, K)]],
                         rows.at[b], sg.at[b])

    def wait_gather(b):
        pltpu.make_async_copy(x_hbm.at[pl.ds(0, K)], rows.at[b], sg.at[b]).wait()

    def start_scatter(s, jc, b):
        pltpu.async_copy(rows.at[b],
                         acc.at[dsts.at[pl.ds(s * CHUNK + jc * K, K)]],
                         ss.at[b], add=True)

    def wait_scatter(b):
        pltpu.make_async_copy(rows.at[b], acc.at[pl.ds(0, K)], ss.at[b]).wait()

    def multiply(s, jc, b):
        def splat(w16, e):
            return lax.gather(
                w16,
                jnp.full((16, 1), e, jnp.int32),
                lax.GatherDimensionNumbers(
                    offset_dims=(), collapsed_slice_dims=(0,),
                    start_index_map=(0,)),
                slice_sizes=(1,),
                mode=lax.GatherScatterMode.PROMISE_IN_BOUNDS,
            )

        def load_pair(g, p):
            r0 = g * 16 + 2 * p
            va = [rows[b, r0, pl.ds(jj * 16, 16)] for jj in range(D // 16)]
            vb = [rows[b, r0 + 1, pl.ds(jj * 16, 16)]
                  for jj in range(D // 16)]
            return va, vb

        def mul_pair(w16, p, va, vb):
            wb0 = splat(w16, 2 * p)
            wb1 = splat(w16, 2 * p + 1)
            return [v * wb0 for v in va], [v * wb1 for v in vb]

        def store_pair(g, p, pa, pb):
            r0 = g * 16 + 2 * p
            for jj in range(D // 16):
                rows[b, r0, pl.ds(jj * 16, 16)] = pa[jj]
            for jj in range(D // 16):
                rows[b, r0 + 1, pl.ds(jj * 16, 16)] = pb[jj]

        @plsc.parallel_loop(0, K // 16, unroll=2)
        def _(g):
            w16 = ws[pl.ds(s * CHUNK + jc * K + g * 16, 16)]
            va, vb = load_pair(g, 0)
            pa, pb = mul_pair(w16, 0, va, vb)
            for p in range(1, 8):
                va, vb = load_pair(g, p)
                store_pair(g, p - 1, pa, pb)
                pa, pb = mul_pair(w16, p, va, vb)
            store_pair(g, 7, pa, pb)

    start_chunk(0)
    plsc.subcore_barrier()
    wait_chunk(0)
    start_gather(0, 0, 0)
    start_gather(0, 1, 1)

    def batch_body(j, _):
        b = lax.rem(j, 3)
        c = lax.div(j, CB)
        jc = j - c * CB
        s = lax.rem(c, 2)
        wait_gather(b)
        multiply(s, jc, b)

        @pl.when(j > 0)
        def _():
            wait_scatter(lax.rem(j + 2, 3))

        @pl.when((jc == 1) & (c + 1 < NCH))
        def _():
            start_chunk(c + 1)

        @pl.when(j + 2 < NB)
        def _():
            j2 = j + 2
            c2 = lax.div(j2, CB)
            jc2 = j2 - c2 * CB

            @pl.when((jc2 == 0) & (c2 >= 1))
            def _():
                wait_chunk(lax.rem(c2, 2))

            start_gather(lax.rem(c2, 2), jc2, lax.rem(j2, 3))

        start_scatter(s, jc, b)
        return 0

    lax.fori_loop(0, NB, batch_body, 0)
    wait_scatter(lax.rem(NB - 1, 3))
    plsc.subcore_barrier()

    @pl.when(sid < NS - 1)
    def _():
        pltpu.sync_copy(acc.at[pl.ds(sid * ZR, ZR)],
                        out_hbm.at[cid, pl.ds(sid * ZR, ZR)])

    @pl.when(sid == NS - 1)
    def _():
        pltpu.sync_copy(acc.at[pl.ds((NS - 1) * ZR, N - (NS - 1) * ZR)],
                        out_hbm.at[cid, pl.ds((NS - 1) * ZR, N - (NS - 1) * ZR)])


_sc_aggregate = pl.kernel(
    _sc_body,
    out_type=jax.ShapeDtypeStruct((NC, N, D), jnp.float32),
    mesh=plsc.VectorSubcoreMesh(core_axis_name="c", subcore_axis_name="s"),
    scratch_types=[
        pltpu.VMEM((2 * CHUNK,), jnp.int32),
        pltpu.VMEM((2 * CHUNK,), jnp.int32),
        pltpu.VMEM((2 * CHUNK,), jnp.float32),
        pltpu.VMEM((3, K, D), jnp.float32),
        pltpu.VMEM_SHARED((N, D), jnp.float32),
        pltpu.SemaphoreType.DMA((3,)),
        pltpu.SemaphoreType.DMA((3,)),
        pltpu.SemaphoreType.DMA,
    ],
)


def _mm_body(pa_ref, pb_ref, w_ref, b_ref, o_ref):
    acc = pa_ref[...] + pb_ref[...]
    o_ref[...] = (
        jnp.dot(acc, w_ref[...], preferred_element_type=jnp.float32)
        + b_ref[...]
    )


_BM = 1000


def _tc_matmul(parts, weights, bias2d):
    return pl.pallas_call(
        _mm_body,
        out_shape=jax.ShapeDtypeStruct((N, D), jnp.float32),
        grid=(N // _BM,),
        in_specs=[
            pl.BlockSpec((_BM, D), lambda i: (i, 0)),
            pl.BlockSpec((_BM, D), lambda i: (i, 0)),
            pl.BlockSpec((D, D), lambda i: (0, 0)),
            pl.BlockSpec((1, D), lambda i: (0, 0)),
        ],
        out_specs=pl.BlockSpec((_BM, D), lambda i: (i, 0)),
    )(parts[0], parts[1], weights, bias2d)


def kernel(feature_map, edge_index, edge_weight, weights, bias):
    src = edge_index[0].astype(jnp.int32)
    dst = edge_index[1].astype(jnp.int32)
    parts = _sc_aggregate(feature_map, src, dst, edge_weight)
    return _tc_matmul(parts, weights, bias.reshape(1, D))


# speedup vs baseline: 1.0373x; 1.0373x over previous
import functools

import jax
import jax.numpy as jnp
from jax import lax
from jax.experimental import pallas as pl
from jax.experimental.pallas import tpu as pltpu
from jax.experimental.pallas import tpu_sc as plsc

N = 10000
E = 320000
D = 128
NC = 2
NS = 16
NW = NC * NS
K = 80
NB = E // (NW * K)
CB = 5
CHUNK = CB * K
NCH = NB // CB
ZR = 624


def _sc_body(x_hbm, src_hbm, dst_hbm, w_hbm, out_hbm,
             srcs, dsts, ws, rowsb, rows, acc, sg, ss, si):
    cid = lax.axis_index("c")
    sid = lax.axis_index("s")
    wid = cid * NS + sid
    ebase = wid * (NB * K)

    zv = jnp.zeros((16,), jnp.float32)

    @plsc.parallel_loop(0, K)
    def _(i):
        for jj in range(D // 16):
            rows[0, i, pl.ds(jj * 16, 16)] = zv

    zb = sid * 625
    for q in range(625 // K):
        pltpu.sync_copy(rows.at[0], acc.at[pl.ds(zb + q * K, K)])
    if 625 % K:
        pltpu.sync_copy(rows.at[0, pl.ds(0, 625 % K)],
                        acc.at[pl.ds(zb + (625 // K) * K, 625 % K)])

    def start_chunk(c):
        s = lax.rem(c, 2)
        eb = ebase + c * CHUNK
        pltpu.async_copy(src_hbm.at[pl.ds(eb, CHUNK)],
                         srcs.at[pl.ds(s * CHUNK, CHUNK)], si)
        pltpu.async_copy(dst_hbm.at[pl.ds(eb, CHUNK)],
                         dsts.at[pl.ds(s * CHUNK, CHUNK)], si)
        pltpu.async_copy(w_hbm.at[pl.ds(eb, CHUNK)],
                         ws.at[pl.ds(s * CHUNK, CHUNK)], si)

    def wait_chunk(s):
        pltpu.make_async_copy(src_hbm.at[pl.ds(0, CHUNK)],
                              srcs.at[pl.ds(s * CHUNK, CHUNK)], si).wait()
        pltpu.make_async_copy(dst_hbm.at[pl.ds(0, CHUNK)],
                              dsts.at[pl.ds(s * CHUNK, CHUNK)], si).wait()
        pltpu.make_async_copy(w_hbm.at[pl.ds(0, CHUNK)],
                              ws.at[pl.ds(s * CHUNK, CHUNK)], si).wait()

    def start_gather(s, jc, b):
        pltpu.async_copy(x_hbm.at[srcs.at[pl.ds(s * CHUNK + jc * K, K)]],
                         rowsb.at[b], sg.at[b])

    def wait_gather(b):
        pltpu.make_async_copy(x_hbm.at[pl.ds(0, K)], rowsb.at[b],
                              sg.at[b]).wait()

    def start_scatter(s, jc, b2):
        pltpu.async_copy(rows.at[b2],
                         acc.at[dsts.at[pl.ds(s * CHUNK + jc * K, K)]],
                         ss.at[b2], add=True)

    def wait_scatter(b2):
        pltpu.make_async_copy(rows.at[b2], acc.at[pl.ds(0, K)],
                              ss.at[b2]).wait()

    def multiply(s, jc, b, b2):
        def splat(w16, e):
            return lax.gather(
                w16,
                jnp.full((16, 1), e, jnp.int32),
                lax.GatherDimensionNumbers(
                    offset_dims=(), collapsed_slice_dims=(0,),
                    start_index_map=(0,)),
                slice_sizes=(1,),
                mode=lax.GatherScatterMode.PROMISE_IN_BOUNDS,
            )

        def load_pair(g, p):
            r0 = g * 16 + 2 * p
            va = [rowsb[b, r0, pl.ds(jj * 16, 16)] for jj in range(D // 32)]
            vb = [rowsb[b, r0 + 1, pl.ds(jj * 16, 16)]
                  for jj in range(D // 32)]
            return va, vb

        def cvt(x):
            lo = lax.bitcast_convert_type(lax.shift_left(x, 16),
                                          jnp.float32)
            hi = lax.bitcast_convert_type(jnp.bitwise_and(x, jnp.int32(-65536)),
                                          jnp.float32)
            return lo, hi

        def mul_pair(w16, p, va, vb):
            wb0 = splat(w16, 2 * p)
            wb1 = splat(w16, 2 * p + 1)
            ua = [cvt(x) for x in va]
            ub = [cvt(x) for x in vb]
            pa = [(lo * wb0, hi * wb0) for (lo, hi) in ua]
            pb = [(lo * wb1, hi * wb1) for (lo, hi) in ub]
            return pa, pb

        def store_pair(g, p, pa, pb):
            r0 = g * 16 + 2 * p
            for jj in range(D // 32):
                rows[b2, r0, pl.ds(jj * 32, 16)] = pa[jj][0]
                rows[b2, r0, pl.ds(jj * 32 + 16, 16)] = pa[jj][1]
            for jj in range(D // 32):
                rows[b2, r0 + 1, pl.ds(jj * 32, 16)] = pb[jj][0]
                rows[b2, r0 + 1, pl.ds(jj * 32 + 16, 16)] = pb[jj][1]

        @plsc.parallel_loop(0, K // 16, unroll=2)
        def _(g):
            w16 = ws[pl.ds(s * CHUNK + jc * K + g * 16, 16)]
            va, vb = load_pair(g, 0)
            pa, pb = mul_pair(w16, 0, va, vb)
            for p in range(1, 8):
                va, vb = load_pair(g, p)
                store_pair(g, p - 1, pa, pb)
                pa, pb = mul_pair(w16, p, va, vb)
            store_pair(g, 7, pa, pb)

    start_chunk(0)
    plsc.subcore_barrier()
    wait_chunk(0)
    start_gather(0, 0, 0)
    start_gather(0, 1, 1)

    def batch_body(j, _):
        b = lax.rem(j, 3)
        b2 = lax.rem(j, 2)
        c = lax.div(j, CB)
        jc = j - c * CB
        s = lax.rem(c, 2)
        wait_gather(b)
        multiply(s, jc, b, b2)

        @pl.when(j > 0)
        def _():
            wait_scatter(lax.rem(j + 1, 2))

        @pl.when((jc == 1) & (c + 1 < NCH))
        def _():
            start_chunk(c + 1)

        @pl.when(j + 2 < NB)
        def _():
            j2 = j + 2
            c2 = lax.div(j2, CB)
            jc2 = j2 - c2 * CB

            @pl.when((jc2 == 0) & (c2 >= 1))
            def _():
                wait_chunk(lax.rem(c2, 2))

            start_gather(lax.rem(c2, 2), jc2, lax.rem(j2, 3))

        start_scatter(s, jc, b2)
        return 0

    lax.fori_loop(0, NB, batch_body, 0)
    wait_scatter(lax.rem(NB - 1, 2))
    plsc.subcore_barrier()

    @pl.when(sid < NS - 1)
    def _():
        pltpu.sync_copy(acc.at[pl.ds(sid * ZR, ZR)],
                        out_hbm.at[cid, pl.ds(sid * ZR, ZR)])

    @pl.when(sid == NS - 1)
    def _():
        pltpu.sync_copy(acc.at[pl.ds((NS - 1) * ZR, N - (NS - 1) * ZR)],
                        out_hbm.at[cid, pl.ds((NS - 1) * ZR, N - (NS - 1) * ZR)])


_sc_aggregate = pl.kernel(
    _sc_body,
    out_type=jax.ShapeDtypeStruct((NC, N, D), jnp.float32),
    mesh=plsc.VectorSubcoreMesh(core_axis_name="c", subcore_axis_name="s"),
    compiler_params=pltpu.CompilerParams(use_tc_tiling_on_sc=False),
    scratch_types=[
        pltpu.VMEM((2 * CHUNK,), jnp.int32),
        pltpu.VMEM((2 * CHUNK,), jnp.int32),
        pltpu.VMEM((2 * CHUNK,), jnp.float32),
        pltpu.VMEM((3, K, D // 2), jnp.int32),
        pltpu.VMEM((2, K, D), jnp.float32),
        pltpu.VMEM_SHARED((N, D), jnp.float32),
        pltpu.SemaphoreType.DMA((3,)),
        pltpu.SemaphoreType.DMA((2,)),
        pltpu.SemaphoreType.DMA,
    ],
)


def _mm_body(pa_ref, pb_ref, w_ref, b_ref, o_ref):
    acc = pa_ref[...] + pb_ref[...]
    o_ref[...] = (
        jnp.dot(acc, w_ref[...], preferred_element_type=jnp.float32)
        + b_ref[...]
    )


_BM = 1000


def _tc_matmul(parts, weights, bias2d):
    return pl.pallas_call(
        _mm_body,
        out_shape=jax.ShapeDtypeStruct((N, D), jnp.float32),
        grid=(N // _BM,),
        in_specs=[
            pl.BlockSpec((_BM, D), lambda i: (i, 0)),
            pl.BlockSpec((_BM, D), lambda i: (i, 0)),
            pl.BlockSpec((D, D), lambda i: (0, 0)),
            pl.BlockSpec((1, D), lambda i: (0, 0)),
        ],
        out_specs=pl.BlockSpec((_BM, D), lambda i: (i, 0)),
    )(parts[0], parts[1], weights, bias2d)


def kernel(feature_map, edge_index, edge_weight, weights, bias):
    src = edge_index[0].astype(jnp.int32)
    dst = edge_index[1].astype(jnp.int32)
    xp = (feature_map.reshape(N, D // 32, 2, 16).swapaxes(2, 3)
          .reshape(N, D).astype(jnp.bfloat16))
    xi = lax.bitcast_convert_type(xp.reshape(N, D // 2, 2), jnp.int32)
    parts = _sc_aggregate(xi, src, dst, edge_weight)
    return _tc_matmul(parts, weights, bias.reshape(1, D))
